# baseline (device time: 83038 ns/iter reference)
import jax
import jax.numpy as jnp
from jax import lax
from jax.experimental import pallas as pl
from jax.experimental.pallas import tpu as pltpu

N_DEV = 4


def kernel(x, w_mat):
    m_per, k = x.shape
    _, n_per = w_mat.shape

    x = x.astype(jnp.bfloat16)
    w = w_mat.astype(jnp.bfloat16)

    def body(x_ref, w_ref, out_ref, comm_ref, send_sems, recv_sems):
        my_pos = lax.axis_index("i")
        left = lax.rem(my_pos + N_DEV - 1, N_DEV)
        right = lax.rem(my_pos + 1, N_DEV)

        barrier_sem = pltpu.get_barrier_semaphore()
        for nbr in [left, right]:
            pl.semaphore_signal(
                barrier_sem, inc=1,
                device_id=(nbr,), device_id_type=pl.DeviceIdType.MESH,
            )
        pl.semaphore_wait(barrier_sem, 2)

        comm_ref[0] = x_ref[...]

        def compute(slot, h):
            origin = lax.rem(my_pos - h + N_DEV, N_DEV)
            y = jnp.dot(comm_ref[slot], w_ref[...],
                        preferred_element_type=jnp.float32)
            y = y * jax.nn.sigmoid(y)
            out_ref[pl.ds(origin * m_per, m_per), :] = y

        for h in range(N_DEV - 1):
            rdma = pltpu.make_async_remote_copy(
                src_ref=comm_ref.at[h],
                dst_ref=comm_ref.at[h + 1],
                send_sem=send_sems.at[h],
                recv_sem=recv_sems.at[h],
                device_id=(right,),
                device_id_type=pl.DeviceIdType.MESH,
            )
            rdma.start()
            compute(h, h)
            rdma.wait()

        compute(N_DEV - 1, N_DEV - 1)

    return pl.pallas_call(
        body,
        out_shape=jax.ShapeDtypeStruct((N_DEV * m_per, n_per), jnp.float32),
        in_specs=[
            pl.BlockSpec(memory_space=pltpu.VMEM),
            pl.BlockSpec(memory_space=pltpu.VMEM),
        ],
        out_specs=pl.BlockSpec(memory_space=pltpu.VMEM),
        scratch_shapes=[
            pltpu.VMEM((N_DEV, m_per, k), jnp.bfloat16),
            pltpu.SemaphoreType.DMA((N_DEV - 1,)),
            pltpu.SemaphoreType.DMA((N_DEV - 1,)),
        ],
        compiler_params=pltpu.CompilerParams(collective_id=0),
    )(x, w)


# device time: 49384 ns/iter; 1.6815x vs baseline; 1.6815x over previous
import jax
import jax.numpy as jnp
from jax import lax
from jax.experimental import pallas as pl
from jax.experimental.pallas import tpu as pltpu

N_DEV = 4


def kernel(x, w_mat):
    m_per, k = x.shape
    _, n_per = w_mat.shape
    m_half = m_per // 2

    x = x.astype(jnp.bfloat16)
    w = w_mat.astype(jnp.bfloat16)

    def body(x_ref, w_ref, out_ref, comm_a, comm_b,
             send_a, recv_a, send_b, recv_b):
        my_pos = lax.axis_index("i")
        left = lax.rem(my_pos + N_DEV - 1, N_DEV)
        right = lax.rem(my_pos + 1, N_DEV)

        barrier_sem = pltpu.get_barrier_semaphore()
        for nbr in [left, right]:
            pl.semaphore_signal(
                barrier_sem, inc=1,
                device_id=(nbr,), device_id_type=pl.DeviceIdType.MESH,
            )
        pl.semaphore_wait(barrier_sem, 2)

        comm_a[0] = x_ref[0:m_half, :]
        comm_b[0] = x_ref[m_half:m_per, :]

        def silu_gemm(src):
            y = jnp.dot(src, w_ref[...], preferred_element_type=jnp.float32)
            return y * jax.nn.sigmoid(y)

        def compute_halves(h):
            o_a = lax.rem(my_pos - h + N_DEV, N_DEV)
            out_ref[pl.ds(o_a * m_per, m_half), :] = silu_gemm(comm_a[h])
            o_b = lax.rem(my_pos + h, N_DEV)
            out_ref[pl.ds(o_b * m_per + m_half, m_half), :] = \
                silu_gemm(comm_b[h])

        for h in range(N_DEV - 1):
            rdma_a = pltpu.make_async_remote_copy(
                src_ref=comm_a.at[h],
                dst_ref=comm_a.at[h + 1],
                send_sem=send_a.at[h],
                recv_sem=recv_a.at[h],
                device_id=(right,),
                device_id_type=pl.DeviceIdType.MESH,
            )
            rdma_b = pltpu.make_async_remote_copy(
                src_ref=comm_b.at[h],
                dst_ref=comm_b.at[h + 1],
                send_sem=send_b.at[h],
                recv_sem=recv_b.at[h],
                device_id=(left,),
                device_id_type=pl.DeviceIdType.MESH,
            )
            rdma_a.start()
            rdma_b.start()
            if h == 0:
                o = my_pos
                out_ref[pl.ds(o * m_per, m_per), :] = silu_gemm(x_ref[...])
            else:
                compute_halves(h)
            rdma_a.wait()
            rdma_b.wait()

        compute_halves(N_DEV - 1)

    return pl.pallas_call(
        body,
        out_shape=jax.ShapeDtypeStruct((N_DEV * m_per, n_per), jnp.float32),
        in_specs=[
            pl.BlockSpec(memory_space=pltpu.VMEM),
            pl.BlockSpec(memory_space=pltpu.VMEM),
        ],
        out_specs=pl.BlockSpec(memory_space=pltpu.VMEM),
        scratch_shapes=[
            pltpu.VMEM((N_DEV, m_half, k), jnp.bfloat16),
            pltpu.VMEM((N_DEV, m_half, k), jnp.bfloat16),
            pltpu.SemaphoreType.DMA((N_DEV - 1,)),
            pltpu.SemaphoreType.DMA((N_DEV - 1,)),
            pltpu.SemaphoreType.DMA((N_DEV - 1,)),
            pltpu.SemaphoreType.DMA((N_DEV - 1,)),
        ],
        compiler_params=pltpu.CompilerParams(collective_id=0),
    )(x, w)


# device time: 46600 ns/iter; 1.7819x vs baseline; 1.0597x over previous
import jax
import jax.numpy as jnp
from jax import lax
from jax.experimental import pallas as pl
from jax.experimental.pallas import tpu as pltpu

N_DEV = 4
N_SUB = 2


def kernel(x, w_mat):
    m_per, k = x.shape
    _, n_per = w_mat.shape
    m_half = m_per // 2
    m_sub = m_half // N_SUB

    def body(x_hbm, w_hbm, out_hbm,
             x_vmem, w_vmem, w_bf, comm_a, comm_b, stage,
             x_sems, w_sem, out_sems,
             send_a, recv_a, send_b, recv_b):
        my_pos = lax.axis_index("i")
        left = lax.rem(my_pos + N_DEV - 1, N_DEV)
        right = lax.rem(my_pos + 1, N_DEV)

        def sub(q):
            return slice(q * m_sub, (q + 1) * m_sub)

        x_copies = []
        for q in range(4):
            c = pltpu.make_async_copy(
                x_hbm.at[sub(q)], x_vmem.at[sub(q)], x_sems.at[q])
            c.start()
            x_copies.append(c)
        w_copy = pltpu.make_async_copy(w_hbm, w_vmem, w_sem)
        w_copy.start()

        barrier_sem = pltpu.get_barrier_semaphore()
        for nbr in [left, right]:
            pl.semaphore_signal(
                barrier_sem, inc=1,
                device_id=(nbr,), device_id_type=pl.DeviceIdType.MESH,
            )
        pl.semaphore_wait(barrier_sem, 2)

        rdma_a = {}
        rdma_b = {}

        def make_a(h, s):
            r = pltpu.make_async_remote_copy(
                src_ref=comm_a.at[h, sub(s)],
                dst_ref=comm_a.at[h + 1, sub(s)],
                send_sem=send_a.at[h, s],
                recv_sem=recv_a.at[h, s],
                device_id=(right,),
                device_id_type=pl.DeviceIdType.MESH,
            )
            rdma_a[(h, s)] = r
            return r

        def make_b(h, s):
            r = pltpu.make_async_remote_copy(
                src_ref=comm_b.at[h, sub(s)],
                dst_ref=comm_b.at[h + 1, sub(s)],
                send_sem=send_b.at[h, s],
                recv_sem=recv_b.at[h, s],
                device_id=(left,),
                device_id_type=pl.DeviceIdType.MESH,
            )
            rdma_b[(h, s)] = r
            return r

        x_copies[0].wait()
        comm_a[0, sub(0)] = x_vmem[sub(0)].astype(jnp.bfloat16)
        make_a(0, 0).start()
        x_copies[2].wait()
        comm_b[0, sub(0)] = x_vmem[sub(2)].astype(jnp.bfloat16)
        make_b(0, 0).start()
        x_copies[1].wait()
        comm_a[0, sub(1)] = x_vmem[sub(1)].astype(jnp.bfloat16)
        make_a(0, 1).start()
        x_copies[3].wait()
        comm_b[0, sub(1)] = x_vmem[sub(3)].astype(jnp.bfloat16)
        make_b(0, 1).start()

        w_copy.wait()
        w_bf[...] = w_vmem[...].astype(jnp.bfloat16)

        def silu_gemm(src):
            y = jnp.dot(src, w_bf[...], preferred_element_type=jnp.float32)
            return y * jax.nn.sigmoid(y)

        out_copies = []

        def store_half(row0, val):
            kidx = len(out_copies)
            slot = kidx % 2
            if kidx >= 2:
                out_copies[kidx - 2].wait()
            stage[slot] = val
            c = pltpu.make_async_copy(
                stage.at[slot],
                out_hbm.at[pl.ds(row0, m_half)],
                out_sems.at[kidx],
            )
            c.start()
            out_copies.append(c)

        store_half(my_pos * m_per, silu_gemm(comm_a[0]))
        store_half(my_pos * m_per + m_half, silu_gemm(comm_b[0]))

        for h in range(1, N_DEV):
            for s in range(N_SUB):
                rdma_a[(h - 1, s)].wait_recv()
                if h < N_DEV - 1:
                    make_a(h, s).start()
                rdma_b[(h - 1, s)].wait_recv()
                if h < N_DEV - 1:
                    make_b(h, s).start()
            o_a = lax.rem(my_pos - h + N_DEV, N_DEV)
            store_half(o_a * m_per, silu_gemm(comm_a[h]))
            o_b = lax.rem(my_pos + h, N_DEV)
            store_half(o_b * m_per + m_half, silu_gemm(comm_b[h]))

        for c in out_copies[-2:]:
            c.wait()
        for d in (rdma_a, rdma_b):
            for r in d.values():
                r.wait_send()

    return pl.pallas_call(
        body,
        out_shape=jax.ShapeDtypeStruct((N_DEV * m_per, n_per), jnp.float32),
        in_specs=[
            pl.BlockSpec(memory_space=pltpu.MemorySpace.HBM),
            pl.BlockSpec(memory_space=pltpu.MemorySpace.HBM),
        ],
        out_specs=pl.BlockSpec(memory_space=pltpu.MemorySpace.HBM),
        scratch_shapes=[
            pltpu.VMEM((m_per, k), jnp.float32),
            pltpu.VMEM((k, n_per), jnp.float32),
            pltpu.VMEM((k, n_per), jnp.bfloat16),
            pltpu.VMEM((N_DEV, m_half, k), jnp.bfloat16),
            pltpu.VMEM((N_DEV, m_half, k), jnp.bfloat16),
            pltpu.VMEM((2, m_half, n_per), jnp.float32),
            pltpu.SemaphoreType.DMA((4,)),
            pltpu.SemaphoreType.DMA,
            pltpu.SemaphoreType.DMA((2 * N_DEV,)),
            pltpu.SemaphoreType.DMA((N_DEV - 1, N_SUB)),
            pltpu.SemaphoreType.DMA((N_DEV - 1, N_SUB)),
            pltpu.SemaphoreType.DMA((N_DEV - 1, N_SUB)),
            pltpu.SemaphoreType.DMA((N_DEV - 1, N_SUB)),
        ],
        compiler_params=pltpu.CompilerParams(collective_id=0),
    )(x, w_mat)
